# baseline (speedup 1.0000x reference)
import functools

import jax
import jax.numpy as jnp
import numpy as np
from jax import lax
from jax.experimental import pallas as pl
from jax.experimental.pallas import tpu as pltpu
from jax.experimental.pallas import tpu_sc as plsc

B = 16384
F = 4096
S = 128
G = 32

B_TC = 12288
B_SC = B - B_TC
BM = 512

NC = 2
NS = 16
NW = NC * NS
ROWS_PER_W = B_SC // NW
R = 8
NBUF = 2
NCHUNK = ROWS_PER_W // R

_mesh = plsc.VectorSubcoreMesh(core_axis_name="c", subcore_axis_name="s")

_scratch = (
    [pltpu.VMEM((R, F), jnp.float32) for _ in range(NBUF)]
    + [pltpu.VMEM((R, S), jnp.float32) for _ in range(NBUF)]
    + [pltpu.SemaphoreType.DMA for _ in range(2 * NBUF)]
)


@functools.partial(
    pl.kernel,
    out_type=jax.ShapeDtypeStruct((B_SC, S), jnp.float32),
    mesh=_mesh,
    compiler_params=pltpu.CompilerParams(needs_layout_passes=False),
    scratch_types=_scratch,
)
def _seg_sum_sc(x_hbm, out_hbm, *scr):
    ins = scr[:NBUF]
    outs = scr[NBUF:2 * NBUF]
    isems = scr[2 * NBUF:3 * NBUF]
    osems = scr[3 * NBUF:]

    wid = lax.axis_index("s") * NC + lax.axis_index("c")
    row0 = B_TC + wid * ROWS_PER_W
    orow0 = wid * ROWS_PER_W

    iota = lax.iota(jnp.int32, 16)
    d33 = iota * 33
    diag = [d33 ^ j for j in range(G)]
    lane0 = iota * 0
    oseg = iota

    def in_src(chunk):
        return x_hbm.at[pl.ds(row0 + chunk * R, R)]

    def out_dst(chunk):
        return out_hbm.at[pl.ds(orow0 + chunk * R, R)]

    def compute(ib, ob):
        def r_body(r, carry):
            row = lane0 + r
            for v in range(8):
                cbase = v * (F // 8)
                acc = plsc.load_gather(ib, [row, cbase + diag[0]])
                for j in range(1, G):
                    acc = acc + plsc.load_gather(ib, [row, cbase + diag[j]])
                plsc.store_scatter(ob, [row, v * 16 + oseg], acc)
            return carry
        lax.fori_loop(0, R, r_body, 0)

    for c in range(NBUF - 1):
        pltpu.async_copy(in_src(c), ins[c], isems[c])

    def step(i, carry):
        for slot in range(NBUF):
            chunk = i * NBUF + slot
            nslot = (slot + NBUF - 1) % NBUF

            @pl.when(chunk + NBUF - 1 < NCHUNK)
            def _():
                pltpu.async_copy(in_src(chunk + NBUF - 1), ins[nslot],
                                 isems[nslot])

            pltpu.make_async_copy(in_src(chunk), ins[slot], isems[slot]).wait()

            @pl.when(chunk >= NBUF)
            def _():
                pltpu.make_async_copy(outs[slot], out_dst(chunk - NBUF),
                                      osems[slot]).wait()

            compute(ins[slot], outs[slot])
            pltpu.async_copy(outs[slot], out_dst(chunk), osems[slot])
        return carry

    lax.fori_loop(0, NCHUNK // NBUF, step, 0)

    for slot in range(NBUF):
        pltpu.make_async_copy(outs[slot], out_dst(NCHUNK - NBUF + slot),
                              osems[slot]).wait()


def _tc_body(x_ref, a_ref, o_ref):
    o_ref[...] = jnp.dot(x_ref[...], a_ref[...],
                         preferred_element_type=jnp.float32)


def _seg_sum_tc(x):
    a = (jnp.arange(F, dtype=jnp.int32)[:, None] // G
         == jnp.arange(S, dtype=jnp.int32)[None, :]).astype(jnp.float32)
    return pl.pallas_call(
        _tc_body,
        grid=(B_TC // BM,),
        in_specs=[
            pl.BlockSpec((BM, F), lambda i: (i, 0)),
            pl.BlockSpec((F, S), lambda i: (0, 0)),
        ],
        out_specs=pl.BlockSpec((BM, S), lambda i: (i, 0)),
        out_shape=jax.ShapeDtypeStruct((B_TC, S), jnp.float32),
    )(x, a)


@jax.jit
def kernel(x):
    out_sc = _seg_sum_sc(x)
    out_tc = _seg_sum_tc(x)
    return jnp.concatenate([out_tc, out_sc], axis=0)


# speedup vs baseline: 1.0318x; 1.0318x over previous
import functools

import jax
import jax.numpy as jnp
import numpy as np
from jax import lax
from jax.experimental import pallas as pl
from jax.experimental.pallas import tpu as pltpu
from jax.experimental.pallas import tpu_sc as plsc

B = 16384
F = 4096
S = 128
G = 32

B_TC = 14336
B_SC = B - B_TC
BM = 512

NC = 2
NS = 16
NW = NC * NS
ROWS_PER_W = B_SC // NW
R = 8
NBUF = 2
NCHUNK = ROWS_PER_W // R

_mesh = plsc.VectorSubcoreMesh(core_axis_name="c", subcore_axis_name="s")

_scratch = (
    [pltpu.VMEM((R, F), jnp.float32) for _ in range(NBUF)]
    + [pltpu.VMEM((R, S), jnp.float32) for _ in range(NBUF)]
    + [pltpu.SemaphoreType.DMA for _ in range(2 * NBUF)]
)


@functools.partial(
    pl.kernel,
    out_type=jax.ShapeDtypeStruct((B_SC, S), jnp.float32),
    mesh=_mesh,
    compiler_params=pltpu.CompilerParams(needs_layout_passes=False),
    scratch_types=_scratch,
)
def _seg_sum_sc(x_hbm, out_hbm, *scr):
    ins = scr[:NBUF]
    outs = scr[NBUF:2 * NBUF]
    isems = scr[2 * NBUF:3 * NBUF]
    osems = scr[3 * NBUF:]

    wid = lax.axis_index("s") * NC + lax.axis_index("c")
    row0 = B_TC + wid * ROWS_PER_W
    orow0 = wid * ROWS_PER_W

    iota = lax.iota(jnp.int32, 16)
    d33 = iota * 33
    diag = [d33 ^ j for j in range(G)]
    lane0 = iota * 0
    oseg = iota

    def in_src(chunk):
        return x_hbm.at[pl.ds(row0 + chunk * R, R)]

    def out_dst(chunk):
        return out_hbm.at[pl.ds(orow0 + chunk * R, R)]

    def compute(ib, ob):
        def r_body(r, carry):
            row = lane0 + r
            for v in range(8):
                cbase = v * (F // 8)
                acc = plsc.load_gather(ib, [row, cbase + diag[0]])
                for j in range(1, G):
                    acc = acc + plsc.load_gather(ib, [row, cbase + diag[j]])
                plsc.store_scatter(ob, [row, v * 16 + oseg], acc)
            return carry
        lax.fori_loop(0, R, r_body, 0)

    for c in range(NBUF - 1):
        pltpu.async_copy(in_src(c), ins[c], isems[c])

    def step(i, carry):
        for slot in range(NBUF):
            chunk = i * NBUF + slot
            nslot = (slot + NBUF - 1) % NBUF

            @pl.when(chunk + NBUF - 1 < NCHUNK)
            def _():
                pltpu.async_copy(in_src(chunk + NBUF - 1), ins[nslot],
                                 isems[nslot])

            pltpu.make_async_copy(in_src(chunk), ins[slot], isems[slot]).wait()

            @pl.when(chunk >= NBUF)
            def _():
                pltpu.make_async_copy(outs[slot], out_dst(chunk - NBUF),
                                      osems[slot]).wait()

            compute(ins[slot], outs[slot])
            pltpu.async_copy(outs[slot], out_dst(chunk), osems[slot])
        return carry

    lax.fori_loop(0, NCHUNK // NBUF, step, 0)

    for slot in range(NBUF):
        pltpu.make_async_copy(outs[slot], out_dst(NCHUNK - NBUF + slot),
                              osems[slot]).wait()


def _tc_body(x_ref, a_ref, o_ref):
    o_ref[...] = jnp.dot(x_ref[...], a_ref[...],
                         preferred_element_type=jnp.float32)


def _seg_sum_tc(x):
    a = (jnp.arange(F, dtype=jnp.int32)[:, None] // G
         == jnp.arange(S, dtype=jnp.int32)[None, :]).astype(jnp.float32)
    return pl.pallas_call(
        _tc_body,
        grid=(B_TC // BM,),
        in_specs=[
            pl.BlockSpec((BM, F), lambda i: (i, 0)),
            pl.BlockSpec((F, S), lambda i: (0, 0)),
        ],
        out_specs=pl.BlockSpec((BM, S), lambda i: (i, 0)),
        out_shape=jax.ShapeDtypeStruct((B_TC, S), jnp.float32),
    )(x, a)


@jax.jit
def kernel(x):
    out_sc = _seg_sum_sc(x)
    out_tc = _seg_sum_tc(x)
    return jnp.concatenate([out_tc, out_sc], axis=0)


# speedup vs baseline: 1.0356x; 1.0037x over previous
import functools

import jax
import jax.numpy as jnp
import numpy as np
from jax import lax
from jax.experimental import pallas as pl
from jax.experimental.pallas import tpu as pltpu
from jax.experimental.pallas import tpu_sc as plsc

B = 16384
F = 4096
S = 128
G = 32

B_TC = 15360
B_SC = B - B_TC
BM = 512

NC = 2
NS = 16
NW = NC * NS
ROWS_PER_W = B_SC // NW
R = 8
NBUF = 2
NCHUNK = ROWS_PER_W // R

_mesh = plsc.VectorSubcoreMesh(core_axis_name="c", subcore_axis_name="s")

_scratch = (
    [pltpu.VMEM((R, F), jnp.float32) for _ in range(NBUF)]
    + [pltpu.VMEM((R, S), jnp.float32) for _ in range(NBUF)]
    + [pltpu.SemaphoreType.DMA for _ in range(2 * NBUF)]
)


@functools.partial(
    pl.kernel,
    out_type=jax.ShapeDtypeStruct((B_SC, S), jnp.float32),
    mesh=_mesh,
    compiler_params=pltpu.CompilerParams(needs_layout_passes=False),
    scratch_types=_scratch,
)
def _seg_sum_sc(x_hbm, out_hbm, *scr):
    ins = scr[:NBUF]
    outs = scr[NBUF:2 * NBUF]
    isems = scr[2 * NBUF:3 * NBUF]
    osems = scr[3 * NBUF:]

    wid = lax.axis_index("s") * NC + lax.axis_index("c")
    row0 = B_TC + wid * ROWS_PER_W
    orow0 = wid * ROWS_PER_W

    iota = lax.iota(jnp.int32, 16)
    d33 = iota * 33
    diag = [d33 ^ j for j in range(G)]
    lane0 = iota * 0
    oseg = iota

    def in_src(chunk):
        return x_hbm.at[pl.ds(row0 + chunk * R, R)]

    def out_dst(chunk):
        return out_hbm.at[pl.ds(orow0 + chunk * R, R)]

    def compute(ib, ob):
        def r_body(r, carry):
            row = lane0 + r
            for v in range(8):
                cbase = v * (F // 8)
                acc = plsc.load_gather(ib, [row, cbase + diag[0]])
                for j in range(1, G):
                    acc = acc + plsc.load_gather(ib, [row, cbase + diag[j]])
                plsc.store_scatter(ob, [row, v * 16 + oseg], acc)
            return carry
        lax.fori_loop(0, R, r_body, 0)

    for c in range(NBUF - 1):
        pltpu.async_copy(in_src(c), ins[c], isems[c])

    def step(i, carry):
        for slot in range(NBUF):
            chunk = i * NBUF + slot
            nslot = (slot + NBUF - 1) % NBUF

            @pl.when(chunk + NBUF - 1 < NCHUNK)
            def _():
                pltpu.async_copy(in_src(chunk + NBUF - 1), ins[nslot],
                                 isems[nslot])

            pltpu.make_async_copy(in_src(chunk), ins[slot], isems[slot]).wait()

            @pl.when(chunk >= NBUF)
            def _():
                pltpu.make_async_copy(outs[slot], out_dst(chunk - NBUF),
                                      osems[slot]).wait()

            compute(ins[slot], outs[slot])
            pltpu.async_copy(outs[slot], out_dst(chunk), osems[slot])
        return carry

    lax.fori_loop(0, NCHUNK // NBUF, step, 0)

    for slot in range(NBUF):
        pltpu.make_async_copy(outs[slot], out_dst(NCHUNK - NBUF + slot),
                              osems[slot]).wait()


def _tc_body(x_ref, a_ref, o_ref):
    o_ref[...] = jnp.dot(x_ref[...], a_ref[...],
                         preferred_element_type=jnp.float32)


def _seg_sum_tc(x):
    a = (jnp.arange(F, dtype=jnp.int32)[:, None] // G
         == jnp.arange(S, dtype=jnp.int32)[None, :]).astype(jnp.float32)
    return pl.pallas_call(
        _tc_body,
        grid=(B_TC // BM,),
        in_specs=[
            pl.BlockSpec((BM, F), lambda i: (i, 0)),
            pl.BlockSpec((F, S), lambda i: (0, 0)),
        ],
        out_specs=pl.BlockSpec((BM, S), lambda i: (i, 0)),
        out_shape=jax.ShapeDtypeStruct((B_TC, S), jnp.float32),
    )(x, a)


@jax.jit
def kernel(x):
    out_sc = _seg_sum_sc(x)
    out_tc = _seg_sum_tc(x)
    return jnp.concatenate([out_tc, out_sc], axis=0)


# speedup vs baseline: 1.0547x; 1.0184x over previous
import functools

import jax
import jax.numpy as jnp
import numpy as np
from jax import lax
from jax.experimental import pallas as pl
from jax.experimental.pallas import tpu as pltpu
from jax.experimental.pallas import tpu_sc as plsc

B = 16384
F = 4096
S = 128
G = 32

B_TC = 15360
B_SC = B - B_TC
BM = 1024

NC = 2
NS = 16
NW = NC * NS
ROWS_PER_W = B_SC // NW
R = 8
NBUF = 2
NCHUNK = ROWS_PER_W // R

_mesh = plsc.VectorSubcoreMesh(core_axis_name="c", subcore_axis_name="s")

_scratch = (
    [pltpu.VMEM((R, F), jnp.float32) for _ in range(NBUF)]
    + [pltpu.VMEM((R, S), jnp.float32) for _ in range(NBUF)]
    + [pltpu.SemaphoreType.DMA for _ in range(2 * NBUF)]
)


@functools.partial(
    pl.kernel,
    out_type=jax.ShapeDtypeStruct((B_SC, S), jnp.float32),
    mesh=_mesh,
    compiler_params=pltpu.CompilerParams(needs_layout_passes=False),
    scratch_types=_scratch,
)
def _seg_sum_sc(x_hbm, out_hbm, *scr):
    ins = scr[:NBUF]
    outs = scr[NBUF:2 * NBUF]
    isems = scr[2 * NBUF:3 * NBUF]
    osems = scr[3 * NBUF:]

    wid = lax.axis_index("s") * NC + lax.axis_index("c")
    row0 = B_TC + wid * ROWS_PER_W
    orow0 = wid * ROWS_PER_W

    iota = lax.iota(jnp.int32, 16)
    d33 = iota * 33
    diag = [d33 ^ j for j in range(G)]
    lane0 = iota * 0
    oseg = iota

    def in_src(chunk):
        return x_hbm.at[pl.ds(row0 + chunk * R, R)]

    def out_dst(chunk):
        return out_hbm.at[pl.ds(orow0 + chunk * R, R)]

    def compute(ib, ob):
        def r_body(r, carry):
            row = lane0 + r
            for v in range(8):
                cbase = v * (F // 8)
                acc = plsc.load_gather(ib, [row, cbase + diag[0]])
                for j in range(1, G):
                    acc = acc + plsc.load_gather(ib, [row, cbase + diag[j]])
                plsc.store_scatter(ob, [row, v * 16 + oseg], acc)
            return carry
        lax.fori_loop(0, R, r_body, 0)

    for c in range(NBUF - 1):
        pltpu.async_copy(in_src(c), ins[c], isems[c])

    def step(i, carry):
        for slot in range(NBUF):
            chunk = i * NBUF + slot
            nslot = (slot + NBUF - 1) % NBUF

            @pl.when(chunk + NBUF - 1 < NCHUNK)
            def _():
                pltpu.async_copy(in_src(chunk + NBUF - 1), ins[nslot],
                                 isems[nslot])

            pltpu.make_async_copy(in_src(chunk), ins[slot], isems[slot]).wait()

            @pl.when(chunk >= NBUF)
            def _():
                pltpu.make_async_copy(outs[slot], out_dst(chunk - NBUF),
                                      osems[slot]).wait()

            compute(ins[slot], outs[slot])
            pltpu.async_copy(outs[slot], out_dst(chunk), osems[slot])
        return carry

    lax.fori_loop(0, NCHUNK // NBUF, step, 0)

    for slot in range(NBUF):
        pltpu.make_async_copy(outs[slot], out_dst(NCHUNK - NBUF + slot),
                              osems[slot]).wait()


def _tc_body(x_ref, a_ref, o_ref):
    o_ref[...] = jnp.dot(x_ref[...], a_ref[...],
                         preferred_element_type=jnp.float32)


def _seg_sum_tc(x):
    a = jnp.asarray(np.arange(F)[:, None] // G == np.arange(S)[None, :],
                    dtype=jnp.float32)
    return pl.pallas_call(
        _tc_body,
        grid=(B_TC // BM,),
        in_specs=[
            pl.BlockSpec((BM, F), lambda i: (i, 0)),
            pl.BlockSpec((F, S), lambda i: (0, 0)),
        ],
        out_specs=pl.BlockSpec((BM, S), lambda i: (i, 0)),
        out_shape=jax.ShapeDtypeStruct((B_TC, S), jnp.float32),
    )(x, a)


@jax.jit
def kernel(x):
    out_sc = _seg_sum_sc(x)
    out_tc = _seg_sum_tc(x)
    return jnp.concatenate([out_tc, out_sc], axis=0)


# speedup vs baseline: 1.0998x; 1.0428x over previous
import functools

import jax
import jax.numpy as jnp
import numpy as np
from jax import lax
from jax.experimental import pallas as pl
from jax.experimental.pallas import tpu as pltpu
from jax.experimental.pallas import tpu_sc as plsc

B = 16384
F = 4096
S = 128
G = 32

B_TC = 15360
B_SC = B - B_TC
BM = 1024

NC = 2
NS = 16
NW = NC * NS
ROWS_PER_W = B_SC // NW
R = 8
NBUF = 2
NCHUNK = ROWS_PER_W // R

_mesh = plsc.VectorSubcoreMesh(core_axis_name="c", subcore_axis_name="s")

_scratch = (
    [pltpu.VMEM((R, F), jnp.float32) for _ in range(NBUF)]
    + [pltpu.VMEM((R, S), jnp.float32) for _ in range(NBUF)]
    + [pltpu.SemaphoreType.DMA for _ in range(2 * NBUF)]
)


@functools.partial(
    pl.kernel,
    out_type=jax.ShapeDtypeStruct((B_SC, S), jnp.float32),
    mesh=_mesh,
    compiler_params=pltpu.CompilerParams(needs_layout_passes=False),
    scratch_types=_scratch,
)
def _seg_sum_sc(x_hbm, out_hbm, *scr):
    ins = scr[:NBUF]
    outs = scr[NBUF:2 * NBUF]
    isems = scr[2 * NBUF:3 * NBUF]
    osems = scr[3 * NBUF:]

    wid = lax.axis_index("s") * NC + lax.axis_index("c")
    row0 = B_TC + wid * ROWS_PER_W
    orow0 = wid * ROWS_PER_W

    iota = lax.iota(jnp.int32, 16)
    d33 = iota * 33
    diag = [d33 ^ j for j in range(G)]
    lane0 = iota * 0
    oseg = iota

    def in_src(chunk):
        return x_hbm.at[pl.ds(row0 + chunk * R, R)]

    def out_dst(chunk):
        return out_hbm.at[pl.ds(orow0 + chunk * R, R)]

    def compute(ib, ob):
        def r_body(r, carry):
            row = lane0 + r
            for v in range(8):
                cbase = v * (F // 8)
                acc = plsc.load_gather(ib, [row, cbase + diag[0]])
                for j in range(1, G):
                    acc = acc + plsc.load_gather(ib, [row, cbase + diag[j]])
                plsc.store_scatter(ob, [row, v * 16 + oseg], acc)
            return carry
        lax.fori_loop(0, R, r_body, 0)

    for c in range(NBUF - 1):
        pltpu.async_copy(in_src(c), ins[c], isems[c])

    def step(i, carry):
        for slot in range(NBUF):
            chunk = i * NBUF + slot
            nslot = (slot + NBUF - 1) % NBUF

            @pl.when(chunk + NBUF - 1 < NCHUNK)
            def _():
                pltpu.async_copy(in_src(chunk + NBUF - 1), ins[nslot],
                                 isems[nslot])

            pltpu.make_async_copy(in_src(chunk), ins[slot], isems[slot]).wait()

            @pl.when(chunk >= NBUF)
            def _():
                pltpu.make_async_copy(outs[slot], out_dst(chunk - NBUF),
                                      osems[slot]).wait()

            compute(ins[slot], outs[slot])
            pltpu.async_copy(outs[slot], out_dst(chunk), osems[slot])
        return carry

    lax.fori_loop(0, NCHUNK // NBUF, step, 0)

    for slot in range(NBUF):
        pltpu.make_async_copy(outs[slot], out_dst(NCHUNK - NBUF + slot),
                              osems[slot]).wait()


def _tc_body(x_ref, a_ref, o_ref):
    o_ref[...] = jnp.dot(x_ref[...], a_ref[...],
                         preferred_element_type=jnp.float32)


def _seg_sum_tc(x):
    a = jnp.asarray(np.arange(F)[:, None] // G == np.arange(S)[None, :],
                    dtype=jnp.float32)
    return pl.pallas_call(
        _tc_body,
        grid=(B_TC // BM,),
        in_specs=[
            pl.BlockSpec((BM, F), lambda i: (i, 0)),
            pl.BlockSpec((F, S), lambda i: (0, 0)),
        ],
        out_specs=pl.BlockSpec((BM, S), lambda i: (i, 0)),
        out_shape=jax.ShapeDtypeStruct((B, S), jnp.float32),
    )(x, a)


@jax.jit
def kernel(x):
    out_sc = _seg_sum_sc(x)
    out_tc = _seg_sum_tc(x)
    return lax.dynamic_update_slice(out_tc, out_sc, (B_TC, 0))
